# baseline (speedup 1.0000x reference)
import functools

import jax
import jax.numpy as jnp
from jax import lax
from jax.experimental import pallas as pl
from jax.experimental.pallas import tpu as pltpu
from jax.experimental.pallas import tpu_sc as plsc

B = 16384
D = 64
NC, NS = 2, 16
NW = NC * NS
BPW = B // NW


def _sc_gather_body(user_hbm, item_hbm, uw_mlp, iw_mlp, uw_mf, iw_mf,
                    ub_mlp, ib_mlp, ub_mf, ib_mf,
                    o_umlp, o_imlp, o_umf, o_imf,
                    o_bu1, o_bi1, o_bu2, o_bi2,
                    idx_u, idx_i, idx_u16, idx_i16, buf_a, buf_b,
                    bb0, bb1, bb2, bb3,
                    sem_a, sem_b, sb0, sb1, sb2, sb3):
    wid = lax.axis_index("s") * NC + lax.axis_index("c")
    base = wid * BPW
    sl = pl.ds(base, BPW)

    pltpu.sync_copy(user_hbm.at[sl], idx_u)
    pltpu.sync_copy(item_hbm.at[sl], idx_i)

    for i in range(BPW // 16):
        s = pl.ds(i * 16, 16)
        idx_u16[s] = lax.shift_right_logical(idx_u[s], 4)
        idx_i16[s] = lax.shift_right_logical(idx_i[s], 4)

    c_b0 = pltpu.async_copy(ub_mlp.at[idx_u16], bb0, sb0)
    c_b1 = pltpu.async_copy(ib_mlp.at[idx_i16], bb1, sb1)
    c_b2 = pltpu.async_copy(ub_mf.at[idx_u16], bb2, sb2)
    c_b3 = pltpu.async_copy(ib_mf.at[idx_i16], bb3, sb3)

    c0 = pltpu.async_copy(uw_mlp.at[idx_u], buf_a, sem_a)
    c1 = pltpu.async_copy(iw_mlp.at[idx_i], buf_b, sem_b)
    c0.wait()
    pltpu.sync_copy(buf_a, o_umlp.at[sl])
    c2 = pltpu.async_copy(uw_mf.at[idx_u], buf_a, sem_a)
    c1.wait()
    pltpu.sync_copy(buf_b, o_imlp.at[sl])
    c3 = pltpu.async_copy(iw_mf.at[idx_i], buf_b, sem_b)
    c2.wait()
    pltpu.sync_copy(buf_a, o_umf.at[sl])
    c3.wait()
    pltpu.sync_copy(buf_b, o_imf.at[sl])

    c_b0.wait()
    pltpu.sync_copy(bb0, o_bu1.at[sl])
    c_b1.wait()
    pltpu.sync_copy(bb1, o_bi1.at[sl])
    c_b2.wait()
    pltpu.sync_copy(bb2, o_bu2.at[sl])
    c_b3.wait()
    pltpu.sync_copy(bb3, o_bi2.at[sl])


_row = jax.ShapeDtypeStruct((B, D), jnp.float32)
_col = jax.ShapeDtypeStruct((B, 16), jnp.float32)


@functools.lru_cache(maxsize=1)
def _make_sc_gather():
  return pl.kernel(
    _sc_gather_body,
    out_type=[_row, _row, _row, _row, _col, _col, _col, _col],
    mesh=plsc.VectorSubcoreMesh(core_axis_name="c", subcore_axis_name="s",
                                num_cores=NC, num_subcores=NS),
    compiler_params=pltpu.CompilerParams(use_tc_tiling_on_sc=False),
    scratch_types=[
        pltpu.VMEM((BPW,), jnp.int32),
        pltpu.VMEM((BPW,), jnp.int32),
        pltpu.VMEM((BPW,), jnp.int32),
        pltpu.VMEM((BPW,), jnp.int32),
        pltpu.VMEM((BPW, D), jnp.float32),
        pltpu.VMEM((BPW, D), jnp.float32),
        pltpu.VMEM((BPW, 16), jnp.float32),
        pltpu.VMEM((BPW, 16), jnp.float32),
        pltpu.VMEM((BPW, 16), jnp.float32),
        pltpu.VMEM((BPW, 16), jnp.float32),
        pltpu.SemaphoreType.DMA,
        pltpu.SemaphoreType.DMA,
        pltpu.SemaphoreType.DMA,
        pltpu.SemaphoreType.DMA,
        pltpu.SemaphoreType.DMA,
        pltpu.SemaphoreType.DMA,
    ],
  )


BLK = 2048


def _dense_body(umlp, imlp, umf, imf, bbu1, bbi1, bbu2, bbi2, lane_u, lane_i,
                w1u, w1i, b1, w2, b2, wa1, wa2, ba, out):
    il16 = lax.broadcasted_iota(jnp.int32, (1, 16), 1)
    mu = lane_u[...] == il16
    mi = lane_i[...] == il16
    bu1 = jnp.sum(jnp.where(mu, bbu1[...], 0.0), axis=1, keepdims=True)
    bi1 = jnp.sum(jnp.where(mi, bbi1[...], 0.0), axis=1, keepdims=True)
    bu2 = jnp.sum(jnp.where(mu, bbu2[...], 0.0), axis=1, keepdims=True)
    bi2 = jnp.sum(jnp.where(mi, bbi2[...], 0.0), axis=1, keepdims=True)
    ue = umlp[...] + bu1
    ie = imlp[...] + bi1
    h = jnp.dot(ue, w1u[...], preferred_element_type=jnp.float32)
    h += jnp.dot(ie, w1i[...], preferred_element_type=jnp.float32)
    h = jnp.maximum(h + b1[...], 0.0)
    h = jnp.dot(h, w2[...], preferred_element_type=jnp.float32)
    h = jnp.maximum(h + b2[...], 0.0)
    mf = (umf[...] + bu2) * (imf[...] + bi2)
    p = jnp.dot(h, wa1[...], preferred_element_type=jnp.float32)
    p += jnp.dot(mf, wa2[...], preferred_element_type=jnp.float32)
    out[...] = p + ba[...]


def _blk(shape):
    return pl.BlockSpec(shape, lambda i: (0,) * len(shape))


_dense = pl.pallas_call(
    _dense_body,
    grid=(B // BLK,),
    in_specs=[
        pl.BlockSpec((BLK, D), lambda i: (i, 0)),
        pl.BlockSpec((BLK, D), lambda i: (i, 0)),
        pl.BlockSpec((BLK, D), lambda i: (i, 0)),
        pl.BlockSpec((BLK, D), lambda i: (i, 0)),
        pl.BlockSpec((BLK, 16), lambda i: (i, 0)),
        pl.BlockSpec((BLK, 16), lambda i: (i, 0)),
        pl.BlockSpec((BLK, 16), lambda i: (i, 0)),
        pl.BlockSpec((BLK, 16), lambda i: (i, 0)),
        pl.BlockSpec((BLK, 1), lambda i: (i, 0)),
        pl.BlockSpec((BLK, 1), lambda i: (i, 0)),
        _blk((D, 32)),
        _blk((D, 32)),
        _blk((1, 32)),
        _blk((32, 16)),
        _blk((1, 16)),
        _blk((16, 1)),
        _blk((D, 1)),
        _blk((1, 1)),
    ],
    out_specs=pl.BlockSpec((BLK, 1), lambda i: (i, 0)),
    out_shape=jax.ShapeDtypeStruct((B, 1), jnp.float32),
)


def kernel(user, item, uw_mlp, ub_mlp, iw_mlp, ib_mlp, uw_mf, ub_mf,
           iw_mf, ib_mf, W1, b1, W2, b2, Wa, ba):
    user = user.astype(jnp.int32)
    item = item.astype(jnp.int32)
    umlp, imlp, umf, imf, bu1, bi1, bu2, bi2 = _make_sc_gather()(
        user, item, uw_mlp, iw_mlp, uw_mf, iw_mf,
        ub_mlp.reshape(-1, 16), ib_mlp.reshape(-1, 16),
        ub_mf.reshape(-1, 16), ib_mf.reshape(-1, 16))
    pred = _dense(
        umlp, imlp, umf, imf, bu1, bi1, bu2, bi2,
        jnp.bitwise_and(user, 15).reshape(B, 1),
        jnp.bitwise_and(item, 15).reshape(B, 1),
        W1[:D], W1[D:], b1.reshape(1, 32), W2, b2.reshape(1, 16),
        Wa[:16], Wa[16:], ba.reshape(1, 1))
    return pred.reshape(-1)


# speedup vs baseline: 1.0024x; 1.0024x over previous
import functools

import jax
import jax.numpy as jnp
from jax import lax
from jax.experimental import pallas as pl
from jax.experimental.pallas import tpu as pltpu
from jax.experimental.pallas import tpu_sc as plsc

B = 16384
D = 64
NC, NS = 2, 16
NW = NC * NS
BPW = B // NW
CH = BPW // 2


def _sc_gather_body(user_hbm, item_hbm, uw_mlp, iw_mlp, uw_mf, iw_mf,
                    ub_mlp, ib_mlp, ub_mf, ib_mf,
                    o_umlp, o_imlp, o_umf, o_imf,
                    o_bu1, o_bi1, o_bu2, o_bi2,
                    idx_u, idx_i, idx_u2, idx_i2, buf_a, buf_b,
                    bb0, bb1, bb2, bb3,
                    sem_a, sem_b, sb0, sb1, sb2, sb3):
    wid = lax.axis_index("s") * NC + lax.axis_index("c")
    base = wid * BPW
    sl = pl.ds(base, BPW)

    pltpu.sync_copy(user_hbm.at[sl], idx_u)
    pltpu.sync_copy(item_hbm.at[sl], idx_i)

    for i in range(BPW // 16):
        s = pl.ds(i * 16, 16)
        idx_u2[s] = lax.shift_right_logical(idx_u[s], 1)
        idx_i2[s] = lax.shift_right_logical(idx_i[s], 1)

    c_b0 = pltpu.async_copy(ub_mlp.at[idx_u], bb0, sb0)
    c_b1 = pltpu.async_copy(ib_mlp.at[idx_i], bb1, sb1)
    c_b2 = pltpu.async_copy(ub_mf.at[idx_u], bb2, sb2)
    c_b3 = pltpu.async_copy(ib_mf.at[idx_i], bb3, sb3)

    plan = [(uw_mlp, idx_u2, o_umlp), (iw_mlp, idx_i2, o_imlp),
            (uw_mf, idx_u2, o_umf), (iw_mf, idx_i2, o_imf)]
    steps = [(t, h) for t in range(4) for h in range(2)]
    bufs = (buf_a, buf_b)
    sems = (sem_a, sem_b)
    copies = [None, None]
    outs = [None, None]
    for n, (t, h) in enumerate(steps):
        table, idx2, out = plan[t]
        p = n % 2
        if copies[p] is not None:
            copies[p].wait()
            pltpu.sync_copy(bufs[p], outs[p])
        copies[p] = pltpu.async_copy(
            table.at[idx2.at[pl.ds(h * CH, CH)]], bufs[p], sems[p])
        outs[p] = out.at[pl.ds(base + h * CH, CH)]
    for p in range(2):
        copies[p].wait()
        pltpu.sync_copy(bufs[p], outs[p])

    c_b0.wait()
    pltpu.sync_copy(bb0, o_bu1.at[sl])
    c_b1.wait()
    pltpu.sync_copy(bb1, o_bi1.at[sl])
    c_b2.wait()
    pltpu.sync_copy(bb2, o_bu2.at[sl])
    c_b3.wait()
    pltpu.sync_copy(bb3, o_bi2.at[sl])


_row = jax.ShapeDtypeStruct((B, 2 * D), jnp.float32)
_col = jax.ShapeDtypeStruct((B,), jnp.float32)


@functools.lru_cache(maxsize=1)
def _make_sc_gather():
  return pl.kernel(
    _sc_gather_body,
    out_type=[_row, _row, _row, _row, _col, _col, _col, _col],
    mesh=plsc.VectorSubcoreMesh(core_axis_name="c", subcore_axis_name="s",
                                num_cores=NC, num_subcores=NS),
    scratch_types=[
        pltpu.VMEM((BPW,), jnp.int32),
        pltpu.VMEM((BPW,), jnp.int32),
        pltpu.VMEM((BPW,), jnp.int32),
        pltpu.VMEM((BPW,), jnp.int32),
        pltpu.VMEM((CH, 2 * D), jnp.float32),
        pltpu.VMEM((CH, 2 * D), jnp.float32),
        pltpu.VMEM((BPW,), jnp.float32),
        pltpu.VMEM((BPW,), jnp.float32),
        pltpu.VMEM((BPW,), jnp.float32),
        pltpu.VMEM((BPW,), jnp.float32),
        pltpu.SemaphoreType.DMA,
        pltpu.SemaphoreType.DMA,
        pltpu.SemaphoreType.DMA,
        pltpu.SemaphoreType.DMA,
        pltpu.SemaphoreType.DMA,
        pltpu.SemaphoreType.DMA,
    ],
  )


BLK = 2048


def _dense_body(umlp, imlp, umf, imf, bu1, bi1, bu2, bi2, par_u, par_i,
                w1u, w1i, b1, w2, b2, wa1, wa2, ba, out):
    pu = par_u[...] == 1
    pi = par_i[...] == 1
    um = umlp[...]
    im = imlp[...]
    uf = umf[...]
    if_ = imf[...]
    ue = jnp.where(pu, um[:, D:], um[:, :D]) + bu1[...]
    ie = jnp.where(pi, im[:, D:], im[:, :D]) + bi1[...]
    h = jnp.dot(ue, w1u[...], preferred_element_type=jnp.float32)
    h += jnp.dot(ie, w1i[...], preferred_element_type=jnp.float32)
    h = jnp.maximum(h + b1[...], 0.0)
    h = jnp.dot(h, w2[...], preferred_element_type=jnp.float32)
    h = jnp.maximum(h + b2[...], 0.0)
    mf = ((jnp.where(pu, uf[:, D:], uf[:, :D]) + bu2[...]) *
          (jnp.where(pi, if_[:, D:], if_[:, :D]) + bi2[...]))
    p = jnp.dot(h, wa1[...], preferred_element_type=jnp.float32)
    p += jnp.dot(mf, wa2[...], preferred_element_type=jnp.float32)
    out[...] = p + ba[...]


def _blk(shape):
    return pl.BlockSpec(shape, lambda i: (0,) * len(shape))


_dense = pl.pallas_call(
    _dense_body,
    grid=(B // BLK,),
    in_specs=[
        pl.BlockSpec((BLK, 2 * D), lambda i: (i, 0)),
        pl.BlockSpec((BLK, 2 * D), lambda i: (i, 0)),
        pl.BlockSpec((BLK, 2 * D), lambda i: (i, 0)),
        pl.BlockSpec((BLK, 2 * D), lambda i: (i, 0)),
        pl.BlockSpec((BLK, 1), lambda i: (i, 0)),
        pl.BlockSpec((BLK, 1), lambda i: (i, 0)),
        pl.BlockSpec((BLK, 1), lambda i: (i, 0)),
        pl.BlockSpec((BLK, 1), lambda i: (i, 0)),
        pl.BlockSpec((BLK, 1), lambda i: (i, 0)),
        pl.BlockSpec((BLK, 1), lambda i: (i, 0)),
        _blk((D, 32)),
        _blk((D, 32)),
        _blk((1, 32)),
        _blk((32, 16)),
        _blk((1, 16)),
        _blk((16, 1)),
        _blk((D, 1)),
        _blk((1, 1)),
    ],
    out_specs=pl.BlockSpec((BLK, 1), lambda i: (i, 0)),
    out_shape=jax.ShapeDtypeStruct((B, 1), jnp.float32),
)


def kernel(user, item, uw_mlp, ub_mlp, iw_mlp, ib_mlp, uw_mf, ub_mf,
           iw_mf, ib_mf, W1, b1, W2, b2, Wa, ba):
    user = user.astype(jnp.int32)
    item = item.astype(jnp.int32)
    umlp, imlp, umf, imf, bu1, bi1, bu2, bi2 = _make_sc_gather()(
        user, item,
        uw_mlp.reshape(-1, 2 * D), iw_mlp.reshape(-1, 2 * D),
        uw_mf.reshape(-1, 2 * D), iw_mf.reshape(-1, 2 * D),
        ub_mlp.reshape(-1), ib_mlp.reshape(-1),
        ub_mf.reshape(-1), ib_mf.reshape(-1))
    pred = _dense(
        umlp, imlp, umf, imf,
        bu1.reshape(B, 1), bi1.reshape(B, 1),
        bu2.reshape(B, 1), bi2.reshape(B, 1),
        jnp.bitwise_and(user, 1).reshape(B, 1),
        jnp.bitwise_and(item, 1).reshape(B, 1),
        W1[:D], W1[D:], b1.reshape(1, 32), W2, b2.reshape(1, 16),
        Wa[:16], Wa[16:], ba.reshape(1, 1))
    return pred.reshape(-1)


# speedup vs baseline: 1.0562x; 1.0537x over previous
import functools

import jax
import jax.numpy as jnp
from jax import lax
from jax.experimental import pallas as pl
from jax.experimental.pallas import tpu as pltpu
from jax.experimental.pallas import tpu_sc as plsc

B = 16384
D = 64
N = 1000000
NC, NS = 2, 16
NW = NC * NS
BPW = B // NW
CH = BPW // 2

TXP = 1024
NGB = 489
H = NGB * TXP


def _tx_body(src_lo, src_hi, out):
    out[:, :D] = jnp.swapaxes(src_lo[...], 0, 1)
    out[:, D:] = jnp.swapaxes(src_hi[...], 0, 1)


_tx = pl.pallas_call(
    _tx_body,
    grid=(NGB,),
    in_specs=[
        pl.BlockSpec((D, TXP), lambda i: (0, i)),
        pl.BlockSpec((D, TXP), lambda i: (0, jnp.minimum(NGB + i, 976))),
    ],
    out_specs=pl.BlockSpec((TXP, 2 * D), lambda i: (i, 0)),
    out_shape=jax.ShapeDtypeStruct((H, 2 * D), jnp.float32),
)


def _sc_gather_body(user_hbm, item_hbm, uw_mlp, iw_mlp, uw_mf, iw_mf,
                    ub_mlp, ib_mlp, ub_mf, ib_mf,
                    o_umlp, o_imlp, o_umf, o_imf,
                    o_bu1, o_bi1, o_bu2, o_bi2,
                    idx_u, idx_i, idx_u2, idx_i2, buf_a, buf_b,
                    bb0, bb1, bb2, bb3,
                    sem_a, sem_b, sb0, sb1, sb2, sb3):
    wid = lax.axis_index("s") * NC + lax.axis_index("c")
    base = wid * BPW
    sl = pl.ds(base, BPW)

    pltpu.sync_copy(user_hbm.at[sl], idx_u)
    pltpu.sync_copy(item_hbm.at[sl], idx_i)

    for i in range(BPW // 16):
        s = pl.ds(i * 16, 16)
        vu = idx_u[s]
        vi = idx_i[s]
        idx_u2[s] = jnp.where(vu >= H, vu - H, vu)
        idx_i2[s] = jnp.where(vi >= H, vi - H, vi)

    c_b0 = pltpu.async_copy(ub_mlp.at[idx_u], bb0, sb0)
    c_b1 = pltpu.async_copy(ib_mlp.at[idx_i], bb1, sb1)
    c_b2 = pltpu.async_copy(ub_mf.at[idx_u], bb2, sb2)
    c_b3 = pltpu.async_copy(ib_mf.at[idx_i], bb3, sb3)

    plan = [(uw_mlp, idx_u2, o_umlp), (iw_mlp, idx_i2, o_imlp),
            (uw_mf, idx_u2, o_umf), (iw_mf, idx_i2, o_imf)]
    steps = [(t, h) for t in range(4) for h in range(2)]
    bufs = (buf_a, buf_b)
    sems = (sem_a, sem_b)
    copies = [None, None]
    outs = [None, None]
    for n, (t, h) in enumerate(steps):
        table, idx2, out = plan[t]
        p = n % 2
        if copies[p] is not None:
            copies[p].wait()
            pltpu.sync_copy(bufs[p], outs[p])
        copies[p] = pltpu.async_copy(
            table.at[idx2.at[pl.ds(h * CH, CH)]], bufs[p], sems[p])
        outs[p] = out.at[pl.ds(base + h * CH, CH)]
    for p in range(2):
        copies[p].wait()
        pltpu.sync_copy(bufs[p], outs[p])

    c_b0.wait()
    pltpu.sync_copy(bb0, o_bu1.at[sl])
    c_b1.wait()
    pltpu.sync_copy(bb1, o_bi1.at[sl])
    c_b2.wait()
    pltpu.sync_copy(bb2, o_bu2.at[sl])
    c_b3.wait()
    pltpu.sync_copy(bb3, o_bi2.at[sl])


_row = jax.ShapeDtypeStruct((B, 2 * D), jnp.float32)
_col = jax.ShapeDtypeStruct((B,), jnp.float32)


@functools.lru_cache(maxsize=1)
def _make_sc_gather():
  return pl.kernel(
    _sc_gather_body,
    out_type=[_row, _row, _row, _row, _col, _col, _col, _col],
    mesh=plsc.VectorSubcoreMesh(core_axis_name="c", subcore_axis_name="s",
                                num_cores=NC, num_subcores=NS),
    scratch_types=[
        pltpu.VMEM((BPW,), jnp.int32),
        pltpu.VMEM((BPW,), jnp.int32),
        pltpu.VMEM((BPW,), jnp.int32),
        pltpu.VMEM((BPW,), jnp.int32),
        pltpu.VMEM((CH, 2 * D), jnp.float32),
        pltpu.VMEM((CH, 2 * D), jnp.float32),
        pltpu.VMEM((BPW,), jnp.float32),
        pltpu.VMEM((BPW,), jnp.float32),
        pltpu.VMEM((BPW,), jnp.float32),
        pltpu.VMEM((BPW,), jnp.float32),
        pltpu.SemaphoreType.DMA,
        pltpu.SemaphoreType.DMA,
        pltpu.SemaphoreType.DMA,
        pltpu.SemaphoreType.DMA,
        pltpu.SemaphoreType.DMA,
        pltpu.SemaphoreType.DMA,
    ],
  )


BLK = 2048


def _dense_body(umlp, imlp, umf, imf, bu1, bi1, bu2, bi2, par_u, par_i,
                w1u, w1i, b1, w2, b2, wa1, wa2, ba, out):
    pu = par_u[...] == 1
    pi = par_i[...] == 1
    um = umlp[...]
    im = imlp[...]
    uf = umf[...]
    if_ = imf[...]
    ue = jnp.where(pu, um[:, D:], um[:, :D]) + bu1[...]
    ie = jnp.where(pi, im[:, D:], im[:, :D]) + bi1[...]
    h = jnp.dot(ue, w1u[...], preferred_element_type=jnp.float32)
    h += jnp.dot(ie, w1i[...], preferred_element_type=jnp.float32)
    h = jnp.maximum(h + b1[...], 0.0)
    h = jnp.dot(h, w2[...], preferred_element_type=jnp.float32)
    h = jnp.maximum(h + b2[...], 0.0)
    mf = ((jnp.where(pu, uf[:, D:], uf[:, :D]) + bu2[...]) *
          (jnp.where(pi, if_[:, D:], if_[:, :D]) + bi2[...]))
    p = jnp.dot(h, wa1[...], preferred_element_type=jnp.float32)
    p += jnp.dot(mf, wa2[...], preferred_element_type=jnp.float32)
    out[...] = p + ba[...]


def _blk(shape):
    return pl.BlockSpec(shape, lambda i: (0,) * len(shape))


_dense = pl.pallas_call(
    _dense_body,
    grid=(B // BLK,),
    in_specs=[
        pl.BlockSpec((BLK, 2 * D), lambda i: (i, 0)),
        pl.BlockSpec((BLK, 2 * D), lambda i: (i, 0)),
        pl.BlockSpec((BLK, 2 * D), lambda i: (i, 0)),
        pl.BlockSpec((BLK, 2 * D), lambda i: (i, 0)),
        pl.BlockSpec((BLK, 1), lambda i: (i, 0)),
        pl.BlockSpec((BLK, 1), lambda i: (i, 0)),
        pl.BlockSpec((BLK, 1), lambda i: (i, 0)),
        pl.BlockSpec((BLK, 1), lambda i: (i, 0)),
        pl.BlockSpec((BLK, 1), lambda i: (i, 0)),
        pl.BlockSpec((BLK, 1), lambda i: (i, 0)),
        _blk((D, 32)),
        _blk((D, 32)),
        _blk((1, 32)),
        _blk((32, 16)),
        _blk((1, 16)),
        _blk((16, 1)),
        _blk((D, 1)),
        _blk((1, 1)),
    ],
    out_specs=pl.BlockSpec((BLK, 1), lambda i: (i, 0)),
    out_shape=jax.ShapeDtypeStruct((B, 1), jnp.float32),
)


def kernel(user, item, uw_mlp, ub_mlp, iw_mlp, ib_mlp, uw_mf, ub_mf,
           iw_mf, ib_mf, W1, b1, W2, b2, Wa, ba):
    user = user.astype(jnp.int32)
    item = item.astype(jnp.int32)
    umlp, imlp, umf, imf, bu1, bi1, bu2, bi2 = _make_sc_gather()(
        user, item,
        _tx(uw_mlp.T, uw_mlp.T), _tx(iw_mlp.T, iw_mlp.T),
        _tx(uw_mf.T, uw_mf.T), _tx(iw_mf.T, iw_mf.T),
        ub_mlp.reshape(-1), ib_mlp.reshape(-1),
        ub_mf.reshape(-1), ib_mf.reshape(-1))
    pred = _dense(
        umlp, imlp, umf, imf,
        bu1.reshape(B, 1), bi1.reshape(B, 1),
        bu2.reshape(B, 1), bi2.reshape(B, 1),
        (user >= H).astype(jnp.int32).reshape(B, 1),
        (item >= H).astype(jnp.int32).reshape(B, 1),
        W1[:D], W1[D:], b1.reshape(1, 32), W2, b2.reshape(1, 16),
        Wa[:16], Wa[16:], ba.reshape(1, 1))
    return pred.reshape(-1)


# speedup vs baseline: 1.7032x; 1.6125x over previous
import functools

import jax
import jax.numpy as jnp
from jax import lax
from jax.experimental import pallas as pl
from jax.experimental.pallas import tpu as pltpu
from jax.experimental.pallas import tpu_sc as plsc

B = 16384
D = 64
N = 1000000
NC, NS = 2, 16
NW = NC * NS
BPW = B // NW
CH = BPW // 2

TXP = 1024
NGB = 489
H = NGB * TXP


def _tx_body(eye, *refs):
    srcs, outs = refs[:8], refs[8:]
    e = eye[...]
    dn = (((0,), (0,)), ((), ()))
    for t in range(4):
        lo = lax.dot_general(srcs[2 * t][...], e, dn,
                             preferred_element_type=jnp.float32)
        hi = lax.dot_general(srcs[2 * t + 1][...], e, dn,
                             preferred_element_type=jnp.float32)
        outs[t][:, :D] = lo
        outs[t][:, D:] = hi


def _lo_spec():
    return pl.BlockSpec((D, TXP), lambda i: (0, i))


def _hi_spec():
    return pl.BlockSpec((D, TXP), lambda i: (0, jnp.minimum(NGB + i, 976)))


_o = jax.ShapeDtypeStruct((H, 2 * D), jnp.float32)

_tx = pl.pallas_call(
    _tx_body,
    grid=(NGB,),
    in_specs=[pl.BlockSpec((D, D), lambda i: (0, 0))]
             + [s for _ in range(4) for s in (_lo_spec(), _hi_spec())],
    out_specs=[pl.BlockSpec((TXP, 2 * D), lambda i: (i, 0))] * 4,
    out_shape=[_o, _o, _o, _o],
)


def _sc_gather_body(user_hbm, item_hbm, uw_mlp, iw_mlp, uw_mf, iw_mf,
                    ub_mlp, ib_mlp, ub_mf, ib_mf,
                    o_umlp, o_imlp, o_umf, o_imf,
                    o_bu1, o_bi1, o_bu2, o_bi2,
                    idx_u, idx_i, idx_u2, idx_i2, buf_a, buf_b,
                    bb0, bb1, bb2, bb3,
                    sem_a, sem_b, sb0, sb1, sb2, sb3):
    wid = lax.axis_index("s") * NC + lax.axis_index("c")
    base = wid * BPW
    sl = pl.ds(base, BPW)

    pltpu.sync_copy(user_hbm.at[sl], idx_u)
    pltpu.sync_copy(item_hbm.at[sl], idx_i)

    for i in range(BPW // 16):
        s = pl.ds(i * 16, 16)
        vu = idx_u[s]
        vi = idx_i[s]
        idx_u2[s] = jnp.where(vu >= H, vu - H, vu)
        idx_i2[s] = jnp.where(vi >= H, vi - H, vi)

    c_b0 = pltpu.async_copy(ub_mlp.at[idx_u], bb0, sb0)
    c_b1 = pltpu.async_copy(ib_mlp.at[idx_i], bb1, sb1)
    c_b2 = pltpu.async_copy(ub_mf.at[idx_u], bb2, sb2)
    c_b3 = pltpu.async_copy(ib_mf.at[idx_i], bb3, sb3)

    plan = [(uw_mlp, idx_u2, o_umlp), (iw_mlp, idx_i2, o_imlp),
            (uw_mf, idx_u2, o_umf), (iw_mf, idx_i2, o_imf)]
    steps = [(t, h) for t in range(4) for h in range(2)]
    bufs = (buf_a, buf_b)
    sems = (sem_a, sem_b)
    copies = [None, None]
    outs = [None, None]
    for n, (t, h) in enumerate(steps):
        table, idx2, out = plan[t]
        p = n % 2
        if copies[p] is not None:
            copies[p].wait()
            pltpu.sync_copy(bufs[p], outs[p])
        copies[p] = pltpu.async_copy(
            table.at[idx2.at[pl.ds(h * CH, CH)]], bufs[p], sems[p])
        outs[p] = out.at[pl.ds(base + h * CH, CH)]
    for p in range(2):
        copies[p].wait()
        pltpu.sync_copy(bufs[p], outs[p])

    c_b0.wait()
    pltpu.sync_copy(bb0, o_bu1.at[sl])
    c_b1.wait()
    pltpu.sync_copy(bb1, o_bi1.at[sl])
    c_b2.wait()
    pltpu.sync_copy(bb2, o_bu2.at[sl])
    c_b3.wait()
    pltpu.sync_copy(bb3, o_bi2.at[sl])


_row = jax.ShapeDtypeStruct((B, 2 * D), jnp.float32)
_col = jax.ShapeDtypeStruct((B,), jnp.float32)


@functools.lru_cache(maxsize=1)
def _make_sc_gather():
  return pl.kernel(
    _sc_gather_body,
    out_type=[_row, _row, _row, _row, _col, _col, _col, _col],
    mesh=plsc.VectorSubcoreMesh(core_axis_name="c", subcore_axis_name="s",
                                num_cores=NC, num_subcores=NS),
    scratch_types=[
        pltpu.VMEM((BPW,), jnp.int32),
        pltpu.VMEM((BPW,), jnp.int32),
        pltpu.VMEM((BPW,), jnp.int32),
        pltpu.VMEM((BPW,), jnp.int32),
        pltpu.VMEM((CH, 2 * D), jnp.float32),
        pltpu.VMEM((CH, 2 * D), jnp.float32),
        pltpu.VMEM((BPW,), jnp.float32),
        pltpu.VMEM((BPW,), jnp.float32),
        pltpu.VMEM((BPW,), jnp.float32),
        pltpu.VMEM((BPW,), jnp.float32),
        pltpu.SemaphoreType.DMA,
        pltpu.SemaphoreType.DMA,
        pltpu.SemaphoreType.DMA,
        pltpu.SemaphoreType.DMA,
        pltpu.SemaphoreType.DMA,
        pltpu.SemaphoreType.DMA,
    ],
  )


BLK = 2048


def _dense_body(umlp, imlp, umf, imf, bu1, bi1, bu2, bi2, par_u, par_i,
                w1u, w1i, b1, w2, b2, wa1, wa2, ba, out):
    pu = par_u[...] == 1
    pi = par_i[...] == 1
    um = umlp[...]
    im = imlp[...]
    uf = umf[...]
    if_ = imf[...]
    ue = jnp.where(pu, um[:, D:], um[:, :D]) + bu1[...]
    ie = jnp.where(pi, im[:, D:], im[:, :D]) + bi1[...]
    h = jnp.dot(ue, w1u[...], preferred_element_type=jnp.float32)
    h += jnp.dot(ie, w1i[...], preferred_element_type=jnp.float32)
    h = jnp.maximum(h + b1[...], 0.0)
    h = jnp.dot(h, w2[...], preferred_element_type=jnp.float32)
    h = jnp.maximum(h + b2[...], 0.0)
    mf = ((jnp.where(pu, uf[:, D:], uf[:, :D]) + bu2[...]) *
          (jnp.where(pi, if_[:, D:], if_[:, :D]) + bi2[...]))
    p = jnp.dot(h, wa1[...], preferred_element_type=jnp.float32)
    p += jnp.dot(mf, wa2[...], preferred_element_type=jnp.float32)
    out[...] = p + ba[...]


def _blk(shape):
    return pl.BlockSpec(shape, lambda i: (0,) * len(shape))


_dense = pl.pallas_call(
    _dense_body,
    grid=(B // BLK,),
    in_specs=[
        pl.BlockSpec((BLK, 2 * D), lambda i: (i, 0)),
        pl.BlockSpec((BLK, 2 * D), lambda i: (i, 0)),
        pl.BlockSpec((BLK, 2 * D), lambda i: (i, 0)),
        pl.BlockSpec((BLK, 2 * D), lambda i: (i, 0)),
        pl.BlockSpec((BLK, 1), lambda i: (i, 0)),
        pl.BlockSpec((BLK, 1), lambda i: (i, 0)),
        pl.BlockSpec((BLK, 1), lambda i: (i, 0)),
        pl.BlockSpec((BLK, 1), lambda i: (i, 0)),
        pl.BlockSpec((BLK, 1), lambda i: (i, 0)),
        pl.BlockSpec((BLK, 1), lambda i: (i, 0)),
        _blk((D, 32)),
        _blk((D, 32)),
        _blk((1, 32)),
        _blk((32, 16)),
        _blk((1, 16)),
        _blk((16, 1)),
        _blk((D, 1)),
        _blk((1, 1)),
    ],
    out_specs=pl.BlockSpec((BLK, 1), lambda i: (i, 0)),
    out_shape=jax.ShapeDtypeStruct((B, 1), jnp.float32),
)


def kernel(user, item, uw_mlp, ub_mlp, iw_mlp, ib_mlp, uw_mf, ub_mf,
           iw_mf, ib_mf, W1, b1, W2, b2, Wa, ba):
    user = user.astype(jnp.int32)
    item = item.astype(jnp.int32)
    eye = jnp.eye(D, dtype=jnp.float32)
    t0, t1, t2, t3 = _tx(eye, uw_mlp.T, uw_mlp.T, iw_mlp.T, iw_mlp.T,
                         uw_mf.T, uw_mf.T, iw_mf.T, iw_mf.T)
    umlp, imlp, umf, imf, bu1, bi1, bu2, bi2 = _make_sc_gather()(
        user, item, t0, t1, t2, t3,
        ub_mlp.reshape(-1), ib_mlp.reshape(-1),
        ub_mf.reshape(-1), ib_mf.reshape(-1))
    pred = _dense(
        umlp, imlp, umf, imf,
        bu1.reshape(B, 1), bi1.reshape(B, 1),
        bu2.reshape(B, 1), bi2.reshape(B, 1),
        (user >= H).astype(jnp.int32).reshape(B, 1),
        (item >= H).astype(jnp.int32).reshape(B, 1),
        W1[:D], W1[D:], b1.reshape(1, 32), W2, b2.reshape(1, 16),
        Wa[:16], Wa[16:], ba.reshape(1, 1))
    return pred.reshape(-1)


# speedup vs baseline: 2.3272x; 1.3664x over previous
import functools

import jax
import jax.numpy as jnp
from jax import lax
from jax.experimental import pallas as pl
from jax.experimental.pallas import tpu as pltpu
from jax.experimental.pallas import tpu_sc as plsc

B = 16384
D = 64
N = 1000000
NC, NS = 2, 16
NW = NC * NS
BPW = B // NW
CH = BPW // 2

TXP = 2048
NGB = 245
H = NGB * TXP
LASTB = (N - 1) // TXP


def _tx_body(eye, *refs):
    srcs, outs = refs[:8], refs[8:]
    e = eye[...]
    dn = (((0,), (0,)), ((), ()))
    for t in range(4):
        x = jnp.concatenate([srcs[2 * t][...], srcs[2 * t + 1][...]], axis=0)
        outs[t][...] = lax.dot_general(x, e, dn,
                                       preferred_element_type=jnp.float32)


def _lo_spec():
    return pl.BlockSpec((D, TXP), lambda i: (0, i))


def _hi_spec():
    return pl.BlockSpec((D, TXP), lambda i: (0, jnp.minimum(NGB + i, LASTB)))


_o = jax.ShapeDtypeStruct((H, 2 * D), jnp.float32)

_tx = pl.pallas_call(
    _tx_body,
    grid=(NGB,),
    in_specs=[pl.BlockSpec((2 * D, 2 * D), lambda i: (0, 0))]
             + [s for _ in range(4) for s in (_lo_spec(), _hi_spec())],
    out_specs=[pl.BlockSpec((TXP, 2 * D), lambda i: (i, 0))] * 4,
    out_shape=[_o, _o, _o, _o],
)


def _sc_gather_body(user_hbm, item_hbm, uw_mlp, iw_mlp, uw_mf, iw_mf,
                    ub_mlp, ib_mlp, ub_mf, ib_mf,
                    o_umlp, o_imlp, o_umf, o_imf,
                    o_bu1, o_bi1, o_bu2, o_bi2,
                    idx_u, idx_i, idx_u2, idx_i2, buf_a, buf_b,
                    bb0, bb1, bb2, bb3,
                    sem_a, sem_b, sb0, sb1, sb2, sb3):
    wid = lax.axis_index("s") * NC + lax.axis_index("c")
    base = wid * BPW
    sl = pl.ds(base, BPW)

    pltpu.sync_copy(user_hbm.at[sl], idx_u)
    pltpu.sync_copy(item_hbm.at[sl], idx_i)

    for i in range(BPW // 16):
        s = pl.ds(i * 16, 16)
        vu = idx_u[s]
        vi = idx_i[s]
        idx_u2[s] = jnp.where(vu >= H, vu - H, vu)
        idx_i2[s] = jnp.where(vi >= H, vi - H, vi)

    c_b0 = pltpu.async_copy(ub_mlp.at[idx_u], bb0, sb0)
    c_b1 = pltpu.async_copy(ib_mlp.at[idx_i], bb1, sb1)
    c_b2 = pltpu.async_copy(ub_mf.at[idx_u], bb2, sb2)
    c_b3 = pltpu.async_copy(ib_mf.at[idx_i], bb3, sb3)

    plan = [(uw_mlp, idx_u2, o_umlp), (iw_mlp, idx_i2, o_imlp),
            (uw_mf, idx_u2, o_umf), (iw_mf, idx_i2, o_imf)]
    steps = [(t, h) for t in range(4) for h in range(2)]
    bufs = (buf_a, buf_b)
    sems = (sem_a, sem_b)
    copies = [None, None]
    outs = [None, None]
    for n, (t, h) in enumerate(steps):
        table, idx2, out = plan[t]
        p = n % 2
        if copies[p] is not None:
            copies[p].wait()
            pltpu.sync_copy(bufs[p], outs[p])
        copies[p] = pltpu.async_copy(
            table.at[idx2.at[pl.ds(h * CH, CH)]], bufs[p], sems[p])
        outs[p] = out.at[pl.ds(base + h * CH, CH)]
    for p in range(2):
        copies[p].wait()
        pltpu.sync_copy(bufs[p], outs[p])

    c_b0.wait()
    pltpu.sync_copy(bb0, o_bu1.at[sl])
    c_b1.wait()
    pltpu.sync_copy(bb1, o_bi1.at[sl])
    c_b2.wait()
    pltpu.sync_copy(bb2, o_bu2.at[sl])
    c_b3.wait()
    pltpu.sync_copy(bb3, o_bi2.at[sl])


_row = jax.ShapeDtypeStruct((B, 2 * D), jnp.float32)
_col = jax.ShapeDtypeStruct((B,), jnp.float32)


@functools.lru_cache(maxsize=1)
def _make_sc_gather():
  return pl.kernel(
    _sc_gather_body,
    out_type=[_row, _row, _row, _row, _col, _col, _col, _col],
    mesh=plsc.VectorSubcoreMesh(core_axis_name="c", subcore_axis_name="s",
                                num_cores=NC, num_subcores=NS),
    scratch_types=[
        pltpu.VMEM((BPW,), jnp.int32),
        pltpu.VMEM((BPW,), jnp.int32),
        pltpu.VMEM((BPW,), jnp.int32),
        pltpu.VMEM((BPW,), jnp.int32),
        pltpu.VMEM((CH, 2 * D), jnp.float32),
        pltpu.VMEM((CH, 2 * D), jnp.float32),
        pltpu.VMEM((BPW,), jnp.float32),
        pltpu.VMEM((BPW,), jnp.float32),
        pltpu.VMEM((BPW,), jnp.float32),
        pltpu.VMEM((BPW,), jnp.float32),
        pltpu.SemaphoreType.DMA,
        pltpu.SemaphoreType.DMA,
        pltpu.SemaphoreType.DMA,
        pltpu.SemaphoreType.DMA,
        pltpu.SemaphoreType.DMA,
        pltpu.SemaphoreType.DMA,
    ],
  )


BLK = 2048


def _dense_body(umlp, imlp, umf, imf, bu1, bi1, bu2, bi2, par_u, par_i,
                w1u, w1i, b1, w2, b2, wa1, wa2, ba, out):
    pu = par_u[...] == 1
    pi = par_i[...] == 1
    um = umlp[...]
    im = imlp[...]
    uf = umf[...]
    if_ = imf[...]
    ue = jnp.where(pu, um[:, D:], um[:, :D]) + bu1[...]
    ie = jnp.where(pi, im[:, D:], im[:, :D]) + bi1[...]
    h = jnp.dot(ue, w1u[...], preferred_element_type=jnp.float32)
    h += jnp.dot(ie, w1i[...], preferred_element_type=jnp.float32)
    h = jnp.maximum(h + b1[...], 0.0)
    h = jnp.dot(h, w2[...], preferred_element_type=jnp.float32)
    h = jnp.maximum(h + b2[...], 0.0)
    mf = ((jnp.where(pu, uf[:, D:], uf[:, :D]) + bu2[...]) *
          (jnp.where(pi, if_[:, D:], if_[:, :D]) + bi2[...]))
    p = jnp.dot(h, wa1[...], preferred_element_type=jnp.float32)
    p += jnp.dot(mf, wa2[...], preferred_element_type=jnp.float32)
    out[...] = p + ba[...]


def _blk(shape):
    return pl.BlockSpec(shape, lambda i: (0,) * len(shape))


_dense = pl.pallas_call(
    _dense_body,
    grid=(B // BLK,),
    in_specs=[
        pl.BlockSpec((BLK, 2 * D), lambda i: (i, 0)),
        pl.BlockSpec((BLK, 2 * D), lambda i: (i, 0)),
        pl.BlockSpec((BLK, 2 * D), lambda i: (i, 0)),
        pl.BlockSpec((BLK, 2 * D), lambda i: (i, 0)),
        pl.BlockSpec((BLK, 1), lambda i: (i, 0)),
        pl.BlockSpec((BLK, 1), lambda i: (i, 0)),
        pl.BlockSpec((BLK, 1), lambda i: (i, 0)),
        pl.BlockSpec((BLK, 1), lambda i: (i, 0)),
        pl.BlockSpec((BLK, 1), lambda i: (i, 0)),
        pl.BlockSpec((BLK, 1), lambda i: (i, 0)),
        _blk((D, 32)),
        _blk((D, 32)),
        _blk((1, 32)),
        _blk((32, 16)),
        _blk((1, 16)),
        _blk((16, 1)),
        _blk((D, 1)),
        _blk((1, 1)),
    ],
    out_specs=pl.BlockSpec((BLK, 1), lambda i: (i, 0)),
    out_shape=jax.ShapeDtypeStruct((B, 1), jnp.float32),
)


def kernel(user, item, uw_mlp, ub_mlp, iw_mlp, ib_mlp, uw_mf, ub_mf,
           iw_mf, ib_mf, W1, b1, W2, b2, Wa, ba):
    user = user.astype(jnp.int32)
    item = item.astype(jnp.int32)
    eye = jnp.eye(2 * D, dtype=jnp.float32)
    t0, t1, t2, t3 = _tx(eye, uw_mlp.T, uw_mlp.T, iw_mlp.T, iw_mlp.T,
                         uw_mf.T, uw_mf.T, iw_mf.T, iw_mf.T)
    umlp, imlp, umf, imf, bu1, bi1, bu2, bi2 = _make_sc_gather()(
        user, item, t0, t1, t2, t3,
        ub_mlp.reshape(-1), ib_mlp.reshape(-1),
        ub_mf.reshape(-1), ib_mf.reshape(-1))
    pred = _dense(
        umlp, imlp, umf, imf,
        bu1.reshape(B, 1), bi1.reshape(B, 1),
        bu2.reshape(B, 1), bi2.reshape(B, 1),
        (user >= H).astype(jnp.int32).reshape(B, 1),
        (item >= H).astype(jnp.int32).reshape(B, 1),
        W1[:D], W1[D:], b1.reshape(1, 32), W2, b2.reshape(1, 16),
        Wa[:16], Wa[16:], ba.reshape(1, 1))
    return pred.reshape(-1)


# speedup vs baseline: 2.8023x; 1.2041x over previous
import functools

import jax
import jax.numpy as jnp
from jax import lax
from jax.experimental import pallas as pl
from jax.experimental.pallas import tpu as pltpu
from jax.experimental.pallas import tpu_sc as plsc

B = 16384
D = 64
N = 1000000
NC, NS = 2, 16
NW = NC * NS
BPW = B // NW
CH = BPW // 2

TXP = 2048
NGB = 123
H4 = NGB * TXP
LASTB = (N - 1) // TXP


def _tx_body(eye, *refs):
    srcs, outs = refs[:16], refs[16:]
    e = eye[...]
    dn = (((0,), (0,)), ((), ()))
    for t in range(4):
        x = jnp.concatenate([srcs[4 * t + s][...] for s in range(4)], axis=0)
        y = lax.dot_general(x, e, dn, preferred_element_type=jnp.float32)
        u_lo = lax.bitcast_convert_type(y[:, :2 * D], jnp.uint32)
        u_hi = lax.bitcast_convert_type(y[:, 2 * D:], jnp.uint32)
        w = (lax.shift_right_logical(u_lo + 0x8000, jnp.uint32(16))
             | ((u_hi + 0x8000) & jnp.uint32(0xFFFF0000)))
        outs[t][...] = lax.bitcast_convert_type(w, jnp.float32)


def _rng_spec(s):
    return pl.BlockSpec(
        (D, TXP), lambda i, s=s: (0, jnp.minimum(s * NGB + i, LASTB)))


_o = jax.ShapeDtypeStruct((H4, 2 * D), jnp.float32)

_tx = pl.pallas_call(
    _tx_body,
    grid=(NGB,),
    in_specs=[pl.BlockSpec((4 * D, 4 * D), lambda i: (0, 0))]
             + [_rng_spec(s) for _ in range(4) for s in range(4)],
    out_specs=[pl.BlockSpec((TXP, 2 * D), lambda i: (i, 0))] * 4,
    out_shape=[_o, _o, _o, _o],
)


def _sc_gather_body(user_hbm, item_hbm, uw_mlp, iw_mlp, uw_mf, iw_mf,
                    ub_mlp, ib_mlp, ub_mf, ib_mf,
                    o_umlp, o_imlp, o_umf, o_imf,
                    o_bu1, o_bi1, o_bu2, o_bi2,
                    idx_u, idx_i, idx_u2, idx_i2, buf_a, buf_b,
                    bb0, bb1, bb2, bb3,
                    sem_a, sem_b, sb0, sb1, sb2, sb3):
    wid = lax.axis_index("s") * NC + lax.axis_index("c")
    base = wid * BPW
    sl = pl.ds(base, BPW)

    pltpu.sync_copy(user_hbm.at[sl], idx_u)
    pltpu.sync_copy(item_hbm.at[sl], idx_i)

    for i in range(BPW // 16):
        s = pl.ds(i * 16, 16)
        vu = idx_u[s]
        vi = idx_i[s]
        vu = jnp.where(vu >= 2 * H4, vu - 2 * H4, vu)
        vi = jnp.where(vi >= 2 * H4, vi - 2 * H4, vi)
        idx_u2[s] = jnp.where(vu >= H4, vu - H4, vu)
        idx_i2[s] = jnp.where(vi >= H4, vi - H4, vi)

    c_b0 = pltpu.async_copy(ub_mlp.at[idx_u], bb0, sb0)
    c_b1 = pltpu.async_copy(ib_mlp.at[idx_i], bb1, sb1)
    c_b2 = pltpu.async_copy(ub_mf.at[idx_u], bb2, sb2)
    c_b3 = pltpu.async_copy(ib_mf.at[idx_i], bb3, sb3)

    plan = [(uw_mlp, idx_u2, o_umlp), (iw_mlp, idx_i2, o_imlp),
            (uw_mf, idx_u2, o_umf), (iw_mf, idx_i2, o_imf)]
    steps = [(t, h) for t in range(4) for h in range(2)]
    bufs = (buf_a, buf_b)
    sems = (sem_a, sem_b)
    copies = [None, None]
    outs = [None, None]
    for n, (t, h) in enumerate(steps):
        table, idx2, out = plan[t]
        p = n % 2
        if copies[p] is not None:
            copies[p].wait()
            pltpu.sync_copy(bufs[p], outs[p])
        copies[p] = pltpu.async_copy(
            table.at[idx2.at[pl.ds(h * CH, CH)]], bufs[p], sems[p])
        outs[p] = out.at[pl.ds(base + h * CH, CH)]
    for p in range(2):
        copies[p].wait()
        pltpu.sync_copy(bufs[p], outs[p])

    c_b0.wait()
    pltpu.sync_copy(bb0, o_bu1.at[sl])
    c_b1.wait()
    pltpu.sync_copy(bb1, o_bi1.at[sl])
    c_b2.wait()
    pltpu.sync_copy(bb2, o_bu2.at[sl])
    c_b3.wait()
    pltpu.sync_copy(bb3, o_bi2.at[sl])


_row = jax.ShapeDtypeStruct((B, 2 * D), jnp.float32)
_col = jax.ShapeDtypeStruct((B,), jnp.float32)


@functools.lru_cache(maxsize=1)
def _make_sc_gather():
  return pl.kernel(
    _sc_gather_body,
    out_type=[_row, _row, _row, _row, _col, _col, _col, _col],
    mesh=plsc.VectorSubcoreMesh(core_axis_name="c", subcore_axis_name="s",
                                num_cores=NC, num_subcores=NS),
    scratch_types=[
        pltpu.VMEM((BPW,), jnp.int32),
        pltpu.VMEM((BPW,), jnp.int32),
        pltpu.VMEM((BPW,), jnp.int32),
        pltpu.VMEM((BPW,), jnp.int32),
        pltpu.VMEM((CH, 2 * D), jnp.float32),
        pltpu.VMEM((CH, 2 * D), jnp.float32),
        pltpu.VMEM((BPW,), jnp.float32),
        pltpu.VMEM((BPW,), jnp.float32),
        pltpu.VMEM((BPW,), jnp.float32),
        pltpu.VMEM((BPW,), jnp.float32),
        pltpu.SemaphoreType.DMA,
        pltpu.SemaphoreType.DMA,
        pltpu.SemaphoreType.DMA,
        pltpu.SemaphoreType.DMA,
        pltpu.SemaphoreType.DMA,
        pltpu.SemaphoreType.DMA,
    ],
  )


BLK = 2048


def _unpack(packed, s):
    w = lax.bitcast_convert_type(packed, jnp.uint32)
    grp = jnp.where((s & 1) == 1, w[:, D:], w[:, :D])
    val = jnp.where(s < 2, lax.shift_left(grp, jnp.uint32(16)),
                    grp & jnp.uint32(0xFFFF0000))
    return lax.bitcast_convert_type(val, jnp.float32)


def _dense_body(umlp, imlp, umf, imf, bu1, bi1, bu2, bi2, sel_u, sel_i,
                w1u, w1i, b1, w2, b2, wa1, wa2, ba, out):
    su = sel_u[...]
    si = sel_i[...]
    ue = _unpack(umlp[...], su) + bu1[...]
    ie = _unpack(imlp[...], si) + bi1[...]
    h = jnp.dot(ue, w1u[...], preferred_element_type=jnp.float32)
    h += jnp.dot(ie, w1i[...], preferred_element_type=jnp.float32)
    h = jnp.maximum(h + b1[...], 0.0)
    h = jnp.dot(h, w2[...], preferred_element_type=jnp.float32)
    h = jnp.maximum(h + b2[...], 0.0)
    mf = ((_unpack(umf[...], su) + bu2[...]) *
          (_unpack(imf[...], si) + bi2[...]))
    p = jnp.dot(h, wa1[...], preferred_element_type=jnp.float32)
    p += jnp.dot(mf, wa2[...], preferred_element_type=jnp.float32)
    out[...] = p + ba[...]


def _blk(shape):
    return pl.BlockSpec(shape, lambda i: (0,) * len(shape))


_dense = pl.pallas_call(
    _dense_body,
    grid=(B // BLK,),
    in_specs=[
        pl.BlockSpec((BLK, 2 * D), lambda i: (i, 0)),
        pl.BlockSpec((BLK, 2 * D), lambda i: (i, 0)),
        pl.BlockSpec((BLK, 2 * D), lambda i: (i, 0)),
        pl.BlockSpec((BLK, 2 * D), lambda i: (i, 0)),
        pl.BlockSpec((BLK, 1), lambda i: (i, 0)),
        pl.BlockSpec((BLK, 1), lambda i: (i, 0)),
        pl.BlockSpec((BLK, 1), lambda i: (i, 0)),
        pl.BlockSpec((BLK, 1), lambda i: (i, 0)),
        pl.BlockSpec((BLK, 1), lambda i: (i, 0)),
        pl.BlockSpec((BLK, 1), lambda i: (i, 0)),
        _blk((D, 32)),
        _blk((D, 32)),
        _blk((1, 32)),
        _blk((32, 16)),
        _blk((1, 16)),
        _blk((16, 1)),
        _blk((D, 1)),
        _blk((1, 1)),
    ],
    out_specs=pl.BlockSpec((BLK, 1), lambda i: (i, 0)),
    out_shape=jax.ShapeDtypeStruct((B, 1), jnp.float32),
)


def kernel(user, item, uw_mlp, ub_mlp, iw_mlp, ib_mlp, uw_mf, ub_mf,
           iw_mf, ib_mf, W1, b1, W2, b2, Wa, ba):
    user = user.astype(jnp.int32)
    item = item.astype(jnp.int32)
    eye = jnp.eye(4 * D, dtype=jnp.float32)
    t0, t1, t2, t3 = _tx(eye,
                         uw_mlp.T, uw_mlp.T, uw_mlp.T, uw_mlp.T,
                         iw_mlp.T, iw_mlp.T, iw_mlp.T, iw_mlp.T,
                         uw_mf.T, uw_mf.T, uw_mf.T, uw_mf.T,
                         iw_mf.T, iw_mf.T, iw_mf.T, iw_mf.T)
    umlp, imlp, umf, imf, bu1, bi1, bu2, bi2 = _make_sc_gather()(
        user, item, t0, t1, t2, t3,
        ub_mlp.reshape(-1), ib_mlp.reshape(-1),
        ub_mf.reshape(-1), ib_mf.reshape(-1))
    pred = _dense(
        umlp, imlp, umf, imf,
        bu1.reshape(B, 1), bi1.reshape(B, 1),
        bu2.reshape(B, 1), bi2.reshape(B, 1),
        ((user >= H4).astype(jnp.int32) + (user >= 2 * H4)
         + (user >= 3 * H4)).reshape(B, 1),
        ((item >= H4).astype(jnp.int32) + (item >= 2 * H4)
         + (item >= 3 * H4)).reshape(B, 1),
        W1[:D], W1[D:], b1.reshape(1, 32), W2, b2.reshape(1, 16),
        Wa[:16], Wa[16:], ba.reshape(1, 1))
    return pred.reshape(-1)


# speedup vs baseline: 2.8609x; 1.0209x over previous
import functools

import jax
import jax.numpy as jnp
from jax import lax
from jax.experimental import pallas as pl
from jax.experimental.pallas import tpu as pltpu
from jax.experimental.pallas import tpu_sc as plsc

B = 16384
D = 64
N = 1000000
NC, NS = 2, 16
NW = NC * NS
BPW = B // NW
CH = BPW // 2

TXP = 4096
NGB = 62
H4 = NGB * TXP
LASTB = (N - 1) // TXP


def _tx_body(eye, *refs):
    srcs, outs = refs[:16], refs[16:]
    e = eye[...]
    dn = (((0,), (0,)), ((), ()))
    for t in range(4):
        x = jnp.concatenate([srcs[4 * t + s][...] for s in range(4)], axis=0)
        y = lax.dot_general(x, e, dn, preferred_element_type=jnp.float32)
        u_lo = lax.bitcast_convert_type(y[:, :2 * D], jnp.uint32)
        u_hi = lax.bitcast_convert_type(y[:, 2 * D:], jnp.uint32)
        w = (lax.shift_right_logical(u_lo + 0x8000, jnp.uint32(16))
             | ((u_hi + 0x8000) & jnp.uint32(0xFFFF0000)))
        outs[t][...] = lax.bitcast_convert_type(w, jnp.float32)


def _rng_spec(s):
    return pl.BlockSpec(
        (D, TXP), lambda i, s=s: (0, jnp.minimum(s * NGB + i, LASTB)))


_o = jax.ShapeDtypeStruct((H4, 2 * D), jnp.float32)

_tx = pl.pallas_call(
    _tx_body,
    grid=(NGB,),
    in_specs=[pl.BlockSpec((4 * D, 4 * D), lambda i: (0, 0))]
             + [_rng_spec(s) for _ in range(4) for s in range(4)],
    out_specs=[pl.BlockSpec((TXP, 2 * D), lambda i: (i, 0))] * 4,
    out_shape=[_o, _o, _o, _o],
)


def _sc_gather_body(user_hbm, item_hbm, uw_mlp, iw_mlp, uw_mf, iw_mf,
                    ub_mlp, ib_mlp, ub_mf, ib_mf,
                    o_umlp, o_imlp, o_umf, o_imf,
                    o_bu1, o_bi1, o_bu2, o_bi2,
                    idx_u, idx_i, idx_u2, idx_i2, buf_a, buf_b,
                    bb0, bb1, bb2, bb3,
                    sem_a, sem_b, sb0, sb1, sb2, sb3):
    wid = lax.axis_index("s") * NC + lax.axis_index("c")
    base = wid * BPW
    sl = pl.ds(base, BPW)

    pltpu.sync_copy(user_hbm.at[sl], idx_u)
    pltpu.sync_copy(item_hbm.at[sl], idx_i)

    for i in range(BPW // 16):
        s = pl.ds(i * 16, 16)
        vu = idx_u[s]
        vi = idx_i[s]
        vu = jnp.where(vu >= 2 * H4, vu - 2 * H4, vu)
        vi = jnp.where(vi >= 2 * H4, vi - 2 * H4, vi)
        idx_u2[s] = jnp.where(vu >= H4, vu - H4, vu)
        idx_i2[s] = jnp.where(vi >= H4, vi - H4, vi)

    c_b0 = pltpu.async_copy(ub_mlp.at[idx_u], bb0, sb0)
    c_b1 = pltpu.async_copy(ib_mlp.at[idx_i], bb1, sb1)
    c_b2 = pltpu.async_copy(ub_mf.at[idx_u], bb2, sb2)
    c_b3 = pltpu.async_copy(ib_mf.at[idx_i], bb3, sb3)

    plan = [(uw_mlp, idx_u2, o_umlp), (iw_mlp, idx_i2, o_imlp),
            (uw_mf, idx_u2, o_umf), (iw_mf, idx_i2, o_imf)]
    steps = [(t, h) for t in range(4) for h in range(2)]
    bufs = (buf_a, buf_b)
    sems = (sem_a, sem_b)
    copies = [None, None]
    outs = [None, None]
    for n, (t, h) in enumerate(steps):
        table, idx2, out = plan[t]
        p = n % 2
        if copies[p] is not None:
            copies[p].wait()
            pltpu.sync_copy(bufs[p], outs[p])
        copies[p] = pltpu.async_copy(
            table.at[idx2.at[pl.ds(h * CH, CH)]], bufs[p], sems[p])
        outs[p] = out.at[pl.ds(base + h * CH, CH)]
    for p in range(2):
        copies[p].wait()
        pltpu.sync_copy(bufs[p], outs[p])

    c_b0.wait()
    pltpu.sync_copy(bb0, o_bu1.at[sl])
    c_b1.wait()
    pltpu.sync_copy(bb1, o_bi1.at[sl])
    c_b2.wait()
    pltpu.sync_copy(bb2, o_bu2.at[sl])
    c_b3.wait()
    pltpu.sync_copy(bb3, o_bi2.at[sl])


_row = jax.ShapeDtypeStruct((B, 2 * D), jnp.float32)
_col = jax.ShapeDtypeStruct((B,), jnp.float32)


@functools.lru_cache(maxsize=1)
def _make_sc_gather():
  return pl.kernel(
    _sc_gather_body,
    out_type=[_row, _row, _row, _row, _col, _col, _col, _col],
    mesh=plsc.VectorSubcoreMesh(core_axis_name="c", subcore_axis_name="s",
                                num_cores=NC, num_subcores=NS),
    scratch_types=[
        pltpu.VMEM((BPW,), jnp.int32),
        pltpu.VMEM((BPW,), jnp.int32),
        pltpu.VMEM((BPW,), jnp.int32),
        pltpu.VMEM((BPW,), jnp.int32),
        pltpu.VMEM((CH, 2 * D), jnp.float32),
        pltpu.VMEM((CH, 2 * D), jnp.float32),
        pltpu.VMEM((BPW,), jnp.float32),
        pltpu.VMEM((BPW,), jnp.float32),
        pltpu.VMEM((BPW,), jnp.float32),
        pltpu.VMEM((BPW,), jnp.float32),
        pltpu.SemaphoreType.DMA,
        pltpu.SemaphoreType.DMA,
        pltpu.SemaphoreType.DMA,
        pltpu.SemaphoreType.DMA,
        pltpu.SemaphoreType.DMA,
        pltpu.SemaphoreType.DMA,
    ],
  )


BLK = 2048


def _unpack(packed, s):
    w = lax.bitcast_convert_type(packed, jnp.uint32)
    grp = jnp.where((s & 1) == 1, w[:, D:], w[:, :D])
    val = jnp.where(s < 2, lax.shift_left(grp, jnp.uint32(16)),
                    grp & jnp.uint32(0xFFFF0000))
    return lax.bitcast_convert_type(val, jnp.float32)


def _dense_body(umlp, imlp, umf, imf, bu1, bi1, bu2, bi2, sel_u, sel_i,
                w1u, w1i, b1, w2, b2, wa1, wa2, ba, out):
    su = sel_u[...]
    si = sel_i[...]
    ue = _unpack(umlp[...], su) + bu1[...]
    ie = _unpack(imlp[...], si) + bi1[...]
    h = jnp.dot(ue, w1u[...], preferred_element_type=jnp.float32)
    h += jnp.dot(ie, w1i[...], preferred_element_type=jnp.float32)
    h = jnp.maximum(h + b1[...], 0.0)
    h = jnp.dot(h, w2[...], preferred_element_type=jnp.float32)
    h = jnp.maximum(h + b2[...], 0.0)
    mf = ((_unpack(umf[...], su) + bu2[...]) *
          (_unpack(imf[...], si) + bi2[...]))
    p = jnp.dot(h, wa1[...], preferred_element_type=jnp.float32)
    p += jnp.dot(mf, wa2[...], preferred_element_type=jnp.float32)
    out[...] = p + ba[...]


def _blk(shape):
    return pl.BlockSpec(shape, lambda i: (0,) * len(shape))


_dense = pl.pallas_call(
    _dense_body,
    grid=(B // BLK,),
    in_specs=[
        pl.BlockSpec((BLK, 2 * D), lambda i: (i, 0)),
        pl.BlockSpec((BLK, 2 * D), lambda i: (i, 0)),
        pl.BlockSpec((BLK, 2 * D), lambda i: (i, 0)),
        pl.BlockSpec((BLK, 2 * D), lambda i: (i, 0)),
        pl.BlockSpec((BLK, 1), lambda i: (i, 0)),
        pl.BlockSpec((BLK, 1), lambda i: (i, 0)),
        pl.BlockSpec((BLK, 1), lambda i: (i, 0)),
        pl.BlockSpec((BLK, 1), lambda i: (i, 0)),
        pl.BlockSpec((BLK, 1), lambda i: (i, 0)),
        pl.BlockSpec((BLK, 1), lambda i: (i, 0)),
        _blk((D, 32)),
        _blk((D, 32)),
        _blk((1, 32)),
        _blk((32, 16)),
        _blk((1, 16)),
        _blk((16, 1)),
        _blk((D, 1)),
        _blk((1, 1)),
    ],
    out_specs=pl.BlockSpec((BLK, 1), lambda i: (i, 0)),
    out_shape=jax.ShapeDtypeStruct((B, 1), jnp.float32),
)


def kernel(user, item, uw_mlp, ub_mlp, iw_mlp, ib_mlp, uw_mf, ub_mf,
           iw_mf, ib_mf, W1, b1, W2, b2, Wa, ba):
    user = user.astype(jnp.int32)
    item = item.astype(jnp.int32)
    eye = jnp.eye(4 * D, dtype=jnp.float32)
    t0, t1, t2, t3 = _tx(eye,
                         uw_mlp.T, uw_mlp.T, uw_mlp.T, uw_mlp.T,
                         iw_mlp.T, iw_mlp.T, iw_mlp.T, iw_mlp.T,
                         uw_mf.T, uw_mf.T, uw_mf.T, uw_mf.T,
                         iw_mf.T, iw_mf.T, iw_mf.T, iw_mf.T)
    umlp, imlp, umf, imf, bu1, bi1, bu2, bi2 = _make_sc_gather()(
        user, item, t0, t1, t2, t3,
        ub_mlp.reshape(-1), ib_mlp.reshape(-1),
        ub_mf.reshape(-1), ib_mf.reshape(-1))
    pred = _dense(
        umlp, imlp, umf, imf,
        bu1.reshape(B, 1), bi1.reshape(B, 1),
        bu2.reshape(B, 1), bi2.reshape(B, 1),
        ((user >= H4).astype(jnp.int32) + (user >= 2 * H4)
         + (user >= 3 * H4)).reshape(B, 1),
        ((item >= H4).astype(jnp.int32) + (item >= 2 * H4)
         + (item >= 3 * H4)).reshape(B, 1),
        W1[:D], W1[D:], b1.reshape(1, 32), W2, b2.reshape(1, 16),
        Wa[:16], Wa[16:], ba.reshape(1, 1))
    return pred.reshape(-1)


# speedup vs baseline: 2.9129x; 1.0182x over previous
import functools

import jax
import jax.numpy as jnp
from jax import lax
from jax.experimental import pallas as pl
from jax.experimental.pallas import tpu as pltpu
from jax.experimental.pallas import tpu_sc as plsc

B = 16384
D = 64
N = 1000000
NC, NS = 2, 16
NW = NC * NS
BPW = B // NW
CH = BPW // 2

TXP = 4096
NGB = 62
H4 = NGB * TXP
LASTB = (N - 1) // TXP
TXQ = TXP // 2
H8 = NGB * TXQ
LASTB8 = (N - 1) // TXQ
DP = 32


def _pack(y):
    u_lo = lax.bitcast_convert_type(y[:, :2 * D], jnp.uint32)
    u_hi = lax.bitcast_convert_type(y[:, 2 * D:], jnp.uint32)
    w = (lax.shift_right_logical(u_lo + 0x8000, jnp.uint32(16))
         | ((u_hi + 0x8000) & jnp.uint32(0xFFFF0000)))
    return lax.bitcast_convert_type(w, jnp.float32)


def _tx_body(eye, wdu, wdi, *refs):
    mlp, mf, outs = refs[:16], refs[16:24], refs[24:]
    dn = (((0,), (0,)), ((), ()))
    for t, wd in ((0, wdu), (1, wdi)):
        x = jnp.concatenate([mlp[8 * t + s][...] for s in range(8)], axis=0)
        y = lax.dot_general(x, wd[...], dn,
                            preferred_element_type=jnp.float32)
        outs[t][...] = _pack(y)
    for t in range(2):
        x = jnp.concatenate([mf[4 * t + s][...] for s in range(4)], axis=0)
        y = lax.dot_general(x, eye[...], dn,
                            preferred_element_type=jnp.float32)
        outs[2 + t][...] = _pack(y)


def _mlp_spec(s):
    return pl.BlockSpec(
        (D, TXQ), lambda i, s=s: (0, jnp.minimum(s * NGB + i, LASTB8)))


def _mf_spec(s):
    return pl.BlockSpec(
        (D, TXP), lambda i, s=s: (0, jnp.minimum(s * NGB + i, LASTB)))


_o8 = jax.ShapeDtypeStruct((H8, 2 * D), jnp.float32)
_o4 = jax.ShapeDtypeStruct((H4, 2 * D), jnp.float32)

_tx = pl.pallas_call(
    _tx_body,
    grid=(NGB,),
    in_specs=[pl.BlockSpec((4 * D, 4 * D), lambda i: (0, 0)),
              pl.BlockSpec((8 * D, 8 * DP), lambda i: (0, 0)),
              pl.BlockSpec((8 * D, 8 * DP), lambda i: (0, 0))]
             + [_mlp_spec(s) for _ in range(2) for s in range(8)]
             + [_mf_spec(s) for _ in range(2) for s in range(4)],
    out_specs=[pl.BlockSpec((TXQ, 2 * D), lambda i: (i, 0))] * 2
              + [pl.BlockSpec((TXP, 2 * D), lambda i: (i, 0))] * 2,
    out_shape=[_o8, _o8, _o4, _o4],
)


def _sc_gather_body(user_hbm, item_hbm, uw_mlp, iw_mlp, uw_mf, iw_mf,
                    ub_mlp, ib_mlp, ub_mf, ib_mf,
                    o_umlp, o_imlp, o_umf, o_imf,
                    o_bu1, o_bi1, o_bu2, o_bi2,
                    idx_u, idx_i, idx_u4, idx_i4, idx_u8, idx_i8,
                    buf_a, buf_b, bb0, bb1, bb2, bb3,
                    sem_a, sem_b, sb0, sb1, sb2, sb3):
    wid = lax.axis_index("s") * NC + lax.axis_index("c")
    base = wid * BPW
    sl = pl.ds(base, BPW)

    pltpu.sync_copy(user_hbm.at[sl], idx_u)
    pltpu.sync_copy(item_hbm.at[sl], idx_i)

    for i in range(BPW // 16):
        s = pl.ds(i * 16, 16)
        vu = idx_u[s]
        vi = idx_i[s]
        vu = jnp.where(vu >= 2 * H4, vu - 2 * H4, vu)
        vi = jnp.where(vi >= 2 * H4, vi - 2 * H4, vi)
        vu = jnp.where(vu >= H4, vu - H4, vu)
        vi = jnp.where(vi >= H4, vi - H4, vi)
        idx_u4[s] = vu
        idx_i4[s] = vi
        idx_u8[s] = jnp.where(vu >= H8, vu - H8, vu)
        idx_i8[s] = jnp.where(vi >= H8, vi - H8, vi)

    c_b0 = pltpu.async_copy(ub_mlp.at[idx_u], bb0, sb0)
    c_b1 = pltpu.async_copy(ib_mlp.at[idx_i], bb1, sb1)
    c_b2 = pltpu.async_copy(ub_mf.at[idx_u], bb2, sb2)
    c_b3 = pltpu.async_copy(ib_mf.at[idx_i], bb3, sb3)

    plan = [(uw_mlp, idx_u8, o_umlp), (iw_mlp, idx_i8, o_imlp),
            (uw_mf, idx_u4, o_umf), (iw_mf, idx_i4, o_imf)]
    steps = [(t, h) for t in range(4) for h in range(2)]
    bufs = (buf_a, buf_b)
    sems = (sem_a, sem_b)
    copies = [None, None]
    outs = [None, None]
    for n, (t, h) in enumerate(steps):
        table, idx2, out = plan[t]
        p = n % 2
        if copies[p] is not None:
            copies[p].wait()
            pltpu.sync_copy(bufs[p], outs[p])
        copies[p] = pltpu.async_copy(
            table.at[idx2.at[pl.ds(h * CH, CH)]], bufs[p], sems[p])
        outs[p] = out.at[pl.ds(base + h * CH, CH)]
    for p in range(2):
        copies[p].wait()
        pltpu.sync_copy(bufs[p], outs[p])

    c_b0.wait()
    pltpu.sync_copy(bb0, o_bu1.at[sl])
    c_b1.wait()
    pltpu.sync_copy(bb1, o_bi1.at[sl])
    c_b2.wait()
    pltpu.sync_copy(bb2, o_bu2.at[sl])
    c_b3.wait()
    pltpu.sync_copy(bb3, o_bi2.at[sl])


_row = jax.ShapeDtypeStruct((B, 2 * D), jnp.float32)
_col = jax.ShapeDtypeStruct((B,), jnp.float32)


@functools.lru_cache(maxsize=1)
def _make_sc_gather():
  return pl.kernel(
    _sc_gather_body,
    out_type=[_row, _row, _row, _row, _col, _col, _col, _col],
    mesh=plsc.VectorSubcoreMesh(core_axis_name="c", subcore_axis_name="s",
                                num_cores=NC, num_subcores=NS),
    scratch_types=[
        pltpu.VMEM((BPW,), jnp.int32),
        pltpu.VMEM((BPW,), jnp.int32),
        pltpu.VMEM((BPW,), jnp.int32),
        pltpu.VMEM((BPW,), jnp.int32),
        pltpu.VMEM((BPW,), jnp.int32),
        pltpu.VMEM((BPW,), jnp.int32),
        pltpu.VMEM((CH, 2 * D), jnp.float32),
        pltpu.VMEM((CH, 2 * D), jnp.float32),
        pltpu.VMEM((BPW,), jnp.float32),
        pltpu.VMEM((BPW,), jnp.float32),
        pltpu.VMEM((BPW,), jnp.float32),
        pltpu.VMEM((BPW,), jnp.float32),
        pltpu.SemaphoreType.DMA,
        pltpu.SemaphoreType.DMA,
        pltpu.SemaphoreType.DMA,
        pltpu.SemaphoreType.DMA,
        pltpu.SemaphoreType.DMA,
        pltpu.SemaphoreType.DMA,
    ],
  )


BLK = 2048


def _unpack(packed, s):
    w = lax.bitcast_convert_type(packed, jnp.uint32)
    grp = jnp.where((s & 1) == 1, w[:, D:], w[:, :D])
    val = jnp.where(s < 2, lax.shift_left(grp, jnp.uint32(16)),
                    grp & jnp.uint32(0xFFFF0000))
    return lax.bitcast_convert_type(val, jnp.float32)


def _unpack8(packed, s):
    w = lax.bitcast_convert_type(packed, jnp.uint32)
    odd = (s & 1) == 1
    c0 = jnp.where(odd, w[:, DP:2 * DP], w[:, :DP])
    c1 = jnp.where(odd, w[:, 3 * DP:], w[:, 2 * DP:3 * DP])
    grp = jnp.where((s & 2) != 0, c1, c0)
    val = jnp.where(s < 4, lax.shift_left(grp, jnp.uint32(16)),
                    grp & jnp.uint32(0xFFFF0000))
    return lax.bitcast_convert_type(val, jnp.float32)


def _dense_body(umlp, imlp, umf, imf, bu1, bi1, bu2, bi2,
                su8, si8, su4, si4,
                cs_u, cs_i, b1, w2, b2, wa1, wa2, ba, out):
    pu = _unpack8(umlp[...], su8[...])
    pi = _unpack8(imlp[...], si8[...])
    h = (pu + pi + bu1[...] * cs_u[...] + bi1[...] * cs_i[...])
    h = jnp.maximum(h + b1[...], 0.0)
    h = jnp.dot(h, w2[...], preferred_element_type=jnp.float32)
    h = jnp.maximum(h + b2[...], 0.0)
    mf = ((_unpack(umf[...], su4[...]) + bu2[...]) *
          (_unpack(imf[...], si4[...]) + bi2[...]))
    p = jnp.dot(h, wa1[...], preferred_element_type=jnp.float32)
    p += jnp.dot(mf, wa2[...], preferred_element_type=jnp.float32)
    out[...] = p + ba[...]


def _blk(shape):
    return pl.BlockSpec(shape, lambda i: (0,) * len(shape))


_dense = pl.pallas_call(
    _dense_body,
    grid=(B // BLK,),
    in_specs=[
        pl.BlockSpec((BLK, 2 * D), lambda i: (i, 0)),
        pl.BlockSpec((BLK, 2 * D), lambda i: (i, 0)),
        pl.BlockSpec((BLK, 2 * D), lambda i: (i, 0)),
        pl.BlockSpec((BLK, 2 * D), lambda i: (i, 0)),
        pl.BlockSpec((BLK, 1), lambda i: (i, 0)),
        pl.BlockSpec((BLK, 1), lambda i: (i, 0)),
        pl.BlockSpec((BLK, 1), lambda i: (i, 0)),
        pl.BlockSpec((BLK, 1), lambda i: (i, 0)),
        pl.BlockSpec((BLK, 1), lambda i: (i, 0)),
        pl.BlockSpec((BLK, 1), lambda i: (i, 0)),
        pl.BlockSpec((BLK, 1), lambda i: (i, 0)),
        pl.BlockSpec((BLK, 1), lambda i: (i, 0)),
        _blk((1, 32)),
        _blk((1, 32)),
        _blk((1, 32)),
        _blk((32, 16)),
        _blk((1, 16)),
        _blk((16, 1)),
        _blk((D, 1)),
        _blk((1, 1)),
    ],
    out_specs=pl.BlockSpec((BLK, 1), lambda i: (i, 0)),
    out_shape=jax.ShapeDtypeStruct((B, 1), jnp.float32),
)


def kernel(user, item, uw_mlp, ub_mlp, iw_mlp, ib_mlp, uw_mf, ub_mf,
           iw_mf, ib_mf, W1, b1, W2, b2, Wa, ba):
    user = user.astype(jnp.int32)
    item = item.astype(jnp.int32)
    eye = jnp.eye(4 * D, dtype=jnp.float32)
    w1u, w1i = W1[:D], W1[D:]
    zb = jnp.zeros((D, DP), jnp.float32)
    wdu = jnp.concatenate(
        [jnp.concatenate([w1u if r == c else zb for c in range(8)], axis=1)
         for r in range(8)], axis=0)
    wdi = jnp.concatenate(
        [jnp.concatenate([w1i if r == c else zb for c in range(8)], axis=1)
         for r in range(8)], axis=0)
    t0, t1, t2, t3 = _tx(eye, wdu, wdi,
                         uw_mlp.T, uw_mlp.T, uw_mlp.T, uw_mlp.T,
                         uw_mlp.T, uw_mlp.T, uw_mlp.T, uw_mlp.T,
                         iw_mlp.T, iw_mlp.T, iw_mlp.T, iw_mlp.T,
                         iw_mlp.T, iw_mlp.T, iw_mlp.T, iw_mlp.T,
                         uw_mf.T, uw_mf.T, uw_mf.T, uw_mf.T,
                         iw_mf.T, iw_mf.T, iw_mf.T, iw_mf.T)
    umlp, imlp, umf, imf, bu1, bi1, bu2, bi2 = _make_sc_gather()(
        user, item, t0, t1, t2, t3,
        ub_mlp.reshape(-1), ib_mlp.reshape(-1),
        ub_mf.reshape(-1), ib_mf.reshape(-1))
    pred = _dense(
        umlp, imlp, umf, imf,
        bu1.reshape(B, 1), bi1.reshape(B, 1),
        bu2.reshape(B, 1), bi2.reshape(B, 1),
        (user // H8).reshape(B, 1), (item // H8).reshape(B, 1),
        (user // H4).reshape(B, 1), (item // H4).reshape(B, 1),
        jnp.sum(w1u, axis=0).reshape(1, 32),
        jnp.sum(w1i, axis=0).reshape(1, 32),
        b1.reshape(1, 32), W2, b2.reshape(1, 16),
        Wa[:16], Wa[16:], ba.reshape(1, 1))
    return pred.reshape(-1)
